# baseline (speedup 1.0000x reference)
import functools

import jax
import jax.numpy as jnp
from jax import lax
from jax.experimental import pallas as pl
from jax.experimental.pallas import tpu as pltpu
from jax.experimental.pallas import tpu_sc as plsc

B = 4
C = 256
A = 9
K = 300
P = 64 * 64 + 32 * 32 + 16 * 16
N = P * A
PT = 512

NPAD = 1280
ROWS_PER_W = NPAD // 32


def _logits_body(x_ref, wpre_ref, bpre_ref, wproj_ref, bproj_ref,
                 logit_ref, prob_ref):
    x = x_ref[...]
    h = jnp.dot(x, wpre_ref[...], preferred_element_type=jnp.float32)
    h = jnp.maximum(h + bpre_ref[...], 0.0)
    l = jnp.dot(h, wproj_ref[...], preferred_element_type=jnp.float32)
    l = l + bproj_ref[...]
    logit_ref[...] = l
    prob_ref[...] = lax.bitcast_convert_type(jax.nn.sigmoid(l), jnp.int32)


def _compute_logits(feats2d, W_pre, b_pre, W_proj, b_proj):
    npos = feats2d.shape[0]
    return pl.pallas_call(
        _logits_body,
        grid=(npos // PT,),
        in_specs=[
            pl.BlockSpec((PT, C), lambda i: (i, 0)),
            pl.BlockSpec((C, C), lambda i: (0, 0)),
            pl.BlockSpec((1, C), lambda i: (0, 0)),
            pl.BlockSpec((C, A), lambda i: (0, 0)),
            pl.BlockSpec((1, A), lambda i: (0, 0)),
        ],
        out_specs=(
            pl.BlockSpec((PT, A), lambda i: (i, 0)),
            pl.BlockSpec((PT, A), lambda i: (i, 0)),
        ),
        out_shape=(
            jax.ShapeDtypeStruct((npos, A), jnp.float32),
            jax.ShapeDtypeStruct((npos, A), jnp.int32),
        ),
    )(feats2d, W_pre.T, b_pre.reshape(1, C), W_proj.T, b_proj.reshape(1, A))


def _topk_body(p_ref, ids_ref, work):
    work[...] = p_ref[...]
    iota2 = lax.broadcasted_iota(jnp.int32, (B, N), 1)
    iotak = lax.broadcasted_iota(jnp.int32, (B, K), 1)
    boff = lax.broadcasted_iota(jnp.int32, (B, 1), 0) * N

    def it(t, ids):
        w = work[...]
        mx = jnp.max(w, axis=1, keepdims=True)
        idx = jnp.min(jnp.where(w == mx, iota2, N), axis=1, keepdims=True)
        ids = jnp.where(iotak == t, boff + idx, ids)
        work[...] = jnp.where(iota2 == idx, -1, w)
        return ids

    ids_ref[...] = lax.fori_loop(0, K, it, jnp.zeros((B, K), jnp.int32))


def _compute_topk_ids(probs_bits):
    return pl.pallas_call(
        _topk_body,
        out_shape=jax.ShapeDtypeStruct((B, K), jnp.int32),
        scratch_shapes=[pltpu.VMEM((B, N), jnp.int32)],
    )(probs_bits)


def _make_sc_gather():
    mesh = plsc.VectorSubcoreMesh(core_axis_name="c", subcore_axis_name="s")

    @functools.partial(
        pl.kernel,
        mesh=mesh,
        out_type=jax.ShapeDtypeStruct((NPAD, C), jnp.float32),
        scratch_types=[
            pltpu.VMEM((ROWS_PER_W,), jnp.int32),
            pltpu.VMEM((ROWS_PER_W, C), jnp.float32),
            pltpu.SemaphoreType.DMA,
        ],
    )
    def gather_k(feats_hbm, idx_hbm, out_hbm, idx_v, rows_v, sem):
        wid = lax.axis_index("s") * 2 + lax.axis_index("c")
        base = wid * ROWS_PER_W
        pltpu.sync_copy(idx_hbm.at[pl.ds(base, ROWS_PER_W)], idx_v)
        pltpu.async_copy(feats_hbm.at[idx_v], rows_v, sem).wait()
        pltpu.sync_copy(rows_v, out_hbm.at[pl.ds(base, ROWS_PER_W)])

    return gather_k


_sc_gather = _make_sc_gather()


def kernel(feat_map0, feat_map1, feat_map2, W_pre, b_pre, W_proj, b_proj):
    feats2d = jnp.concatenate(
        [jnp.transpose(fm.reshape(B, C, -1), (0, 2, 1))
         for fm in (feat_map0, feat_map1, feat_map2)], axis=1).reshape(B * P, C)

    logits, probs_bits = _compute_logits(feats2d, W_pre, b_pre, W_proj, b_proj)
    sel_logits = logits.reshape(B, N)

    sel_ids = _compute_topk_ids(probs_bits.reshape(B, N)).reshape(-1)

    feat_ids = sel_ids // A
    idx_pad = jnp.zeros((NPAD,), jnp.int32).at[:B * K].set(feat_ids)
    sel_feats = _sc_gather(feats2d, idx_pad)[:B * K]

    return sel_logits, sel_ids, sel_feats
